# baseline (speedup 1.0000x reference)
import functools
import math

import jax
import jax.numpy as jnp
from jax import lax
from jax.experimental import pallas as pl
from jax.experimental.pallas import tpu as pltpu
from jax.experimental.pallas import tpu_sc as plsc

_NUM_CORES = 2
_NUM_SUBCORES = 16
_LANES = 16
_NUM_WORKERS = _NUM_CORES * _NUM_SUBCORES

_CHUNK = 64
_NBUF = 2
_PREFETCH = 1


@functools.lru_cache(maxsize=None)
def _make_lookup(vocab, d_model, batch):
  assert batch % _NUM_WORKERS == 0
  b_per_w = batch // _NUM_WORKERS
  assert b_per_w % _CHUNK == 0
  n_chunks = b_per_w // _CHUNK
  n_vecs = d_model // _LANES
  scale = math.sqrt(float(d_model))

  mesh = plsc.VectorSubcoreMesh(core_axis_name="c", subcore_axis_name="s")

  @functools.partial(
      pl.kernel,
      mesh=mesh,
      out_type=jax.ShapeDtypeStruct((batch, d_model), jnp.float32),
      scratch_types=[
          pltpu.VMEM((b_per_w,), jnp.int32),
          pltpu.VMEM((_NBUF, _CHUNK, d_model), jnp.float32),
          [pltpu.SemaphoreType.DMA for _ in range(_NBUF)],
          [pltpu.SemaphoreType.DMA for _ in range(_NBUF)],
      ],
  )
  def lookup(table_hbm, idx_hbm, out_hbm, idx_v, rows_v, gsems, ssems):
    wid = lax.axis_index("s") * _NUM_CORES + lax.axis_index("c")
    base = wid * b_per_w
    pltpu.sync_copy(idx_hbm.at[pl.ds(base, b_per_w)], idx_v)

    def gather_copy(c, buf):
      return pltpu.make_async_copy(
          table_hbm.at[idx_v.at[pl.ds(c * _CHUNK, _CHUNK)]],
          rows_v.at[buf],
          gsems[buf],
      )

    def store_copy(c, buf):
      return pltpu.make_async_copy(
          rows_v.at[buf],
          out_hbm.at[pl.ds(base + c * _CHUNK, _CHUNK)],
          ssems[buf],
      )

    gather_copy(0, 0).start()
    if _PREFETCH > 1:
      gather_copy(1, 1).start()

    @pl.loop(0, n_chunks // _NBUF)
    def per_round(g):
      c0 = g * _NBUF
      for b in range(_NBUF):
        c = c0 + b
        gather_copy(c, b).wait()
        bn = (b + _PREFETCH) % _NBUF
        cn = c + _PREFETCH

        @pl.when(cn < n_chunks)
        def _():
          @pl.when(cn >= _NBUF)
          def _():
            store_copy(cn - _NBUF, bn).wait()

          gather_copy(cn, bn).start()

        @pl.loop(0, _CHUNK, step=2, unroll=2)
        def scale_row(r):
          for j in range(n_vecs):
            sl = pl.ds(j * _LANES, _LANES)
            rows_v[b, r, sl] = rows_v[b, r, sl] * scale
          for j in range(n_vecs):
            sl = pl.ds(j * _LANES, _LANES)
            rows_v[b, r + 1, sl] = rows_v[b, r + 1, sl] * scale
        store_copy(c, b).start()

    for b in range(_NBUF):
      store_copy(n_chunks - _NBUF + b, b).wait()

  return lookup


def kernel(x, table):
  vocab, d_model = table.shape
  x_flat = x.reshape(-1).astype(jnp.int32)
  out = _make_lookup(vocab, d_model, x_flat.shape[0])(table, x_flat)
  return out.reshape(*x.shape, d_model)


# speedup vs baseline: 1.1126x; 1.1126x over previous
import functools
import math

import jax
import jax.numpy as jnp
from jax import lax
from jax.experimental import pallas as pl
from jax.experimental.pallas import tpu as pltpu
from jax.experimental.pallas import tpu_sc as plsc

_NUM_CORES = 2
_NUM_SUBCORES = 16
_LANES = 16
_NUM_WORKERS = _NUM_CORES * _NUM_SUBCORES

_CHUNK = 64
_NBUF = 2
_PREFETCH = 1


@functools.lru_cache(maxsize=None)
def _make_lookup(vocab, d_model, batch):
  assert batch % _NUM_WORKERS == 0
  b_per_w = batch // _NUM_WORKERS
  assert b_per_w % _CHUNK == 0
  n_chunks = b_per_w // _CHUNK
  n_vecs = d_model // _LANES
  scale = math.sqrt(float(d_model))

  mesh = plsc.VectorSubcoreMesh(core_axis_name="c", subcore_axis_name="s")

  @functools.partial(
      pl.kernel,
      mesh=mesh,
      out_type=jax.ShapeDtypeStruct((batch, d_model), jnp.float32),
      scratch_types=[
          pltpu.VMEM((n_chunks, _CHUNK), jnp.int32),
          pltpu.VMEM((_NBUF, _CHUNK, d_model), jnp.float32),
          [pltpu.SemaphoreType.DMA for _ in range(_NBUF)],
          [pltpu.SemaphoreType.DMA for _ in range(_NBUF)],
      ],
  )
  def lookup(table_hbm, idx_hbm, out_hbm, idx_v, rows_v, gsems, ssems):
    wid = lax.axis_index("s") * _NUM_CORES + lax.axis_index("c")
    base = wid * b_per_w
    pltpu.sync_copy(idx_hbm.at[wid], idx_v)

    def gather_copy(c, buf):
      return pltpu.make_async_copy(
          table_hbm.at[idx_v.at[c]],
          rows_v.at[buf],
          gsems[buf],
      )

    def store_copy(c, buf):
      return pltpu.make_async_copy(
          rows_v.at[buf],
          out_hbm.at[pl.ds(base + c * _CHUNK, _CHUNK)],
          ssems[buf],
      )

    gather_copy(0, 0).start()
    if _PREFETCH > 1:
      gather_copy(1, 1).start()

    @pl.loop(0, n_chunks // _NBUF)
    def per_round(g):
      c0 = g * _NBUF
      for b in range(_NBUF):
        c = c0 + b
        gather_copy(c, b).wait()
        bn = (b + _PREFETCH) % _NBUF
        cn = c + _PREFETCH

        @pl.when(cn < n_chunks)
        def _():
          @pl.when(cn >= _NBUF)
          def _():
            store_copy(cn - _NBUF, bn).wait()

          gather_copy(cn, bn).start()

        def scale_row(r, _):
          for j in range(n_vecs):
            sl = pl.ds(j * _LANES, _LANES)
            rows_v[b, r, sl] = rows_v[b, r, sl] * scale
          return _

        lax.fori_loop(0, _CHUNK, scale_row, 0)
        store_copy(c, b).start()

    for b in range(_NBUF):
      store_copy(n_chunks - _NBUF + b, b).wait()

  return lookup


def kernel(x, table):
  vocab, d_model = table.shape
  batch = x.size
  b_per_w = batch // _NUM_WORKERS
  x_3d = x.reshape(_NUM_WORKERS, b_per_w // _CHUNK, _CHUNK).astype(jnp.int32)
  out = _make_lookup(vocab, d_model, batch)(table, x_3d)
  return out.reshape(*x.shape, d_model)
